# baseline (speedup 1.0000x reference)
import math

import jax
import jax.numpy as jnp
from jax import lax
from jax.experimental import pallas as pl
from jax.experimental.pallas import tpu as pltpu
from jax.experimental.pallas import tpu_sc as plsc

_SIZE = 100000
_PAD = 0
_SMOOTHING = 0.1
_CONF = 1.0 - _SMOOTHING
_EPS = _SMOOTHING / (_SIZE - 2)
_C = _CONF * math.log(_CONF) + (_SIZE - 2) * _EPS * math.log(_EPS)

_BR = 16
_L = 16
_NW = 32


def _tc_main_body(x_ref, t_ref, o_ref):
    x = x_ref[...]
    t = t_ref[0, 0, :]
    rowsum = jnp.sum(x, axis=1)
    col0 = x[:, 0]
    cols = jax.lax.broadcasted_iota(jnp.int32, x.shape, 1)
    g = jnp.sum(jnp.where(cols == t[:, None], x, 0.0), axis=1)
    per_row = jnp.where(
        t != _PAD,
        _EPS * col0 - _EPS * rowsum - (_CONF - _EPS) * g,
        0.0,
    )
    partial = jnp.sum(per_row)

    @pl.when(pl.program_id(0) == 0)
    def _():
        o_ref[0, 0] = 0.0

    o_ref[0, 0] += partial


def _sc_part(target, n):
    rows_pt = n // _NW
    mesh = plsc.VectorSubcoreMesh(core_axis_name="c", subcore_axis_name="s")

    @pl.kernel(
        mesh=mesh,
        out_type=jax.ShapeDtypeStruct((_NW, _L), jnp.float32),
        scratch_types=[
            pltpu.VMEM((rows_pt,), jnp.int32),
            pltpu.VMEM((_L,), jnp.float32),
        ],
    )
    def sc_kernel(t_hbm, out_hbm, tgt_v, tot_v):
        wid = lax.axis_index("s") * 2 + lax.axis_index("c")
        pltpu.sync_copy(t_hbm.at[pl.ds(wid * rows_pt, rows_pt)], tgt_v)
        cnt = jnp.zeros((_L,), jnp.float32)
        for g in range(rows_pt // _L):
            t16 = tgt_v[pl.ds(g * _L, _L)]
            cnt = cnt + jnp.where(t16 != _PAD, 1.0, 0.0)
        tot_v[...] = _C * cnt
        pltpu.sync_copy(tot_v, out_hbm.at[wid])

    return sc_kernel(target)


def _sc_probe(x1d):
    mesh = plsc.VectorSubcoreMesh(core_axis_name="c", subcore_axis_name="s")

    @pl.kernel(
        mesh=mesh,
        out_type=jax.ShapeDtypeStruct((_NW, _L), jnp.float32),
        scratch_types=[
            pltpu.VMEM((1408,), jnp.float32),
            pltpu.VMEM((_L,), jnp.float32),
        ],
    )
    def sc_kernel(x_hbm, out_hbm, buf, tot_v):
        wid = lax.axis_index("s") * 2 + lax.axis_index("c")
        acc = jnp.zeros((_L,), jnp.float32)
        for c in range(4):
            off = (512 + wid * 48) * _SIZE + c * 1408
            pltpu.sync_copy(x_hbm.at[pl.ds(off, 1408)], buf)
            for j in range(1408 // _L):
                acc = acc + buf[pl.ds(j * _L, _L)]
        tot_v[...] = acc
        pltpu.sync_copy(tot_v, out_hbm.at[wid])

    return sc_kernel(x1d)


def kernel(x, target):
    n, v = x.shape
    t32 = target.astype(jnp.int32)
    t3 = t32.reshape(n // _BR, 1, _BR)

    sc_out = _sc_part(t32, n) + _sc_probe(x.reshape(-1)) * 0.0

    dense = pl.pallas_call(
        _tc_main_body,
        grid=(n // _BR,),
        in_specs=[
            pl.BlockSpec((_BR, v), lambda i: (i, 0)),
            pl.BlockSpec((1, 1, _BR), lambda i: (i, 0, 0)),
        ],
        out_specs=pl.BlockSpec(memory_space=pltpu.SMEM),
        out_shape=jax.ShapeDtypeStruct((1, 1), jnp.float32),
    )(x, t3)

    return dense[0, 0] + jnp.sum(sc_out)


# speedup vs baseline: 2.0973x; 2.0973x over previous
import math

import jax
import jax.numpy as jnp
from jax import lax
from jax.experimental import pallas as pl
from jax.experimental.pallas import tpu as pltpu
from jax.experimental.pallas import tpu_sc as plsc

_SIZE = 100000
_PAD = 0
_SMOOTHING = 0.1
_CONF = 1.0 - _SMOOTHING
_EPS = _SMOOTHING / (_SIZE - 2)
_C = _CONF * math.log(_CONF) + (_SIZE - 2) * _EPS * math.log(_EPS)

_BR = 16
_L = 16
_NW = 32
_W = 128


def _tc_main_body(x_ref, t_ref, o_ref):
    x = x_ref[...]
    t = t_ref[0, 0, :]
    rowsum = jnp.sum(x, axis=1)
    cols = jax.lax.broadcasted_iota(jnp.int32, x.shape, 1)
    g = jnp.sum(jnp.where(cols == t[:, None], x, 0.0), axis=1)
    per_row = jnp.where(
        t != _PAD,
        -_EPS * rowsum - (_CONF - _EPS) * g,
        0.0,
    )
    partial = jnp.sum(per_row)

    @pl.when(pl.program_id(0) == 0)
    def _():
        o_ref[0, 0] = 0.0

    o_ref[0, 0] += partial


def _sc_part(xs1d, target, n):
    rows_pt = n // _NW
    mesh = plsc.VectorSubcoreMesh(core_axis_name="c", subcore_axis_name="s")

    @pl.kernel(
        mesh=mesh,
        out_type=jax.ShapeDtypeStruct((_NW, _L), jnp.float32),
        scratch_types=[
            pltpu.VMEM((rows_pt * _W,), jnp.float32),
            pltpu.VMEM((rows_pt,), jnp.int32),
            pltpu.VMEM((_L,), jnp.float32),
        ],
    )
    def sc_kernel(x_hbm, t_hbm, out_hbm, buf, tgt_v, tot_v):
        wid = lax.axis_index("s") * 2 + lax.axis_index("c")
        pltpu.sync_copy(t_hbm.at[pl.ds(wid * rows_pt, rows_pt)], tgt_v)
        pltpu.sync_copy(x_hbm.at[pl.ds(wid * rows_pt * _W, rows_pt * _W)],
                        buf)
        lanes = lax.iota(jnp.int32, _L)
        cnt = jnp.zeros((_L,), jnp.float32)
        x0acc = jnp.zeros((_L,), jnp.float32)
        for g in range(rows_pt // _L):
            t16 = tgt_v[pl.ds(g * _L, _L)]
            cnt = cnt + jnp.where(t16 != _PAD, 1.0, 0.0)
            for r in range(_L):
                m = t16[r] != _PAD
                v16 = buf[pl.ds((g * _L + r) * _W, _L)]
                sel0 = jnp.where(m, 0, -1)
                pick0 = lanes == jnp.full((_L,), sel0)
                x0acc = x0acc + jnp.where(pick0, v16, 0.0)
        tot_v[...] = _EPS * x0acc + _C * cnt
        pltpu.sync_copy(tot_v, out_hbm.at[wid])

    return sc_kernel(xs1d, target)


def kernel(x, target):
    n, v = x.shape
    t32 = target.astype(jnp.int32)
    t3 = t32.reshape(n // _BR, 1, _BR)

    xs1d = lax.slice(x, (0, 0), (n, _W)).reshape(-1)
    sc_out = _sc_part(xs1d, t32, n)

    dense = pl.pallas_call(
        _tc_main_body,
        grid=(n // _BR,),
        in_specs=[
            pl.BlockSpec((_BR, v), lambda i: (i, 0)),
            pl.BlockSpec((1, 1, _BR), lambda i: (i, 0, 0)),
        ],
        out_specs=pl.BlockSpec(memory_space=pltpu.SMEM),
        out_shape=jax.ShapeDtypeStruct((1, 1), jnp.float32),
    )(x, t3)

    return dense[0, 0] + jnp.sum(sc_out)
